# baseline (speedup 1.0000x reference)
import functools

import jax
import jax.numpy as jnp
from jax import lax
from jax.experimental import pallas as pl
from jax.experimental.pallas import tpu as pltpu
from jax.experimental.pallas import tpu_sc as plsc

N = 10000
E = 320000
D = 128
G = 64
EPS = 0.1

NC = 2
NS = 16
DH = D // NC
EW = E // NS
K = 80
NCHUNK = EW // K
NR = N // 4
NACC = NR + NS

_mesh = plsc.VectorSubcoreMesh(core_axis_name="c", subcore_axis_name="s")


@functools.partial(
    pl.kernel,
    out_type=jax.ShapeDtypeStruct((NC, N), jnp.float32),
    mesh=_mesh,
    scratch_types=[
        pltpu.VMEM((K,), jnp.int32),
        pltpu.VMEM((K,), jnp.float32),
        pltpu.VMEM((N,), jnp.float32),
        pltpu.VMEM_SHARED((N,), jnp.float32),
    ],
)
def _deg_kernel(dst_hbm, out_hbm, idx_v, ones_v, stage_v, acc):
    cid = lax.axis_index("c")
    sid = lax.axis_index("s")

    def fill_ones(i, _):
        ones_v[pl.ds(i * 16, 16)] = jnp.full((16,), 1.0, jnp.float32)
        return 0

    lax.fori_loop(0, K // 16, fill_ones, 0)

    def fill_zero(i, _):
        stage_v[pl.ds(i * 16, 16)] = jnp.zeros((16,), jnp.float32)
        return 0

    lax.fori_loop(0, N // 16, fill_zero, 0)

    @pl.when(sid == 0)
    def _():
        pltpu.sync_copy(stage_v, acc)

    plsc.subcore_barrier()

    base = (cid * NS + sid) * (E // (NC * NS))

    def chunk(g, _):
        pltpu.sync_copy(dst_hbm.at[pl.ds(base + g * K, K)], idx_v)
        pltpu.sync_copy(ones_v, acc.at[idx_v], add=True)
        return 0

    lax.fori_loop(0, E // (NC * NS) // K, chunk, 0)
    plsc.subcore_barrier()

    @pl.when(sid == 0)
    def _():
        pltpu.sync_copy(acc, stage_v)
        pltpu.sync_copy(stage_v, out_hbm.at[cid])


@functools.partial(
    pl.kernel,
    out_type=jax.ShapeDtypeStruct((NC, NACC, D), jnp.float32),
    mesh=_mesh,
    scratch_types=[
        pltpu.VMEM((16,), jnp.int32),
        pltpu.VMEM((K,), jnp.int32),
        pltpu.VMEM((K,), jnp.int32),
        pltpu.VMEM((K,), jnp.int32),
        pltpu.VMEM((K, D), jnp.float32),
        pltpu.VMEM((K, D), jnp.float32),
        pltpu.VMEM((K, D), jnp.float32),
        pltpu.VMEM((K, D), jnp.float32),
        pltpu.VMEM_SHARED((NACC, D), jnp.float32),
        pltpu.SemaphoreType.DMA,
    ],
)
def _edge_kernel(a_hbm, b_hbm, c01_hbm, lsel_hbm, zero_hbm, dst_hbm,
                 src_hbm, out_hbm, lv, dsti, srci, dsts, ar, br, cr, tr, acc,
                 sem):
    cid = lax.axis_index("c")
    sid = lax.axis_index("s")

    pltpu.sync_copy(lsel_hbm, lv)
    lvv = lv[...]
    lsel = lvv[0]
    nhalf = lvv[1]

    @pl.when(sid == 0)
    def _():
        pltpu.sync_copy(zero_hbm, acc)

    plsc.subcore_barrier()

    base = sid * EW
    node0 = nhalf * (2 * NR) + cid * NR
    trash = NR + sid

    def chunk(g, _):
        e0 = base + g * K
        pltpu.sync_copy(dst_hbm.at[pl.ds(e0, K)], dsti)
        pltpu.sync_copy(src_hbm.at[pl.ds(e0, K)], srci)
        pltpu.async_copy(a_hbm.at[dsti], ar, sem).wait()
        pltpu.async_copy(b_hbm.at[srci], br, sem).wait()
        pltpu.sync_copy(c01_hbm.at[lsel].at[pl.ds(e0, K)], cr)

        def remap(m, _):
            sl = pl.ds(m * 16, 16)
            k = dsti[sl] - node0
            ok = (k >= 0) & (k < NR)
            dsts[sl] = jnp.where(ok, k, trash)
            return 0

        lax.fori_loop(0, K // 16, remap, 0)

        def comp(r, _):
            for j in range(D // 16):
                sl = pl.ds(j * 16, 16)
                tr[r, sl] = jnp.maximum(ar[r, sl] + br[r, sl] + cr[r, sl],
                                        0.0)
            return 0

        lax.fori_loop(0, K, comp, 0)
        pltpu.sync_copy(tr, acc.at[dsts], add=True)
        return 0

    lax.fori_loop(0, NCHUNK, chunk, 0)
    plsc.subcore_barrier()

    @pl.when(sid == 0)
    def _():
        pltpu.sync_copy(acc, out_hbm.at[cid])


@functools.partial(
    pl.kernel,
    out_type=jax.ShapeDtypeStruct((NC, 4 * G, D), jnp.float32),
    mesh=_mesh,
    scratch_types=[
        pltpu.VMEM((K,), jnp.int32),
        pltpu.VMEM((K,), jnp.int32),
        pltpu.VMEM((K,), jnp.int32),
        pltpu.VMEM((K, D), jnp.float32),
        pltpu.VMEM((K, D), jnp.float32),
        pltpu.VMEM((K, DH), jnp.float32),
        pltpu.VMEM((K, D), jnp.float32),
        pltpu.VMEM_SHARED((4 * G, D), jnp.float32),
        pltpu.SemaphoreType.DMA,
    ],
)
def _edge_pool_kernel(a_hbm, b_hbm, c_hbm, dst_hbm, src_hbm, gidx_hbm,
                      zero_hbm, out_hbm, dsti, srci, gidx, ar, br, cr, tr,
                      acc, sem):
    cid = lax.axis_index("c")
    sid = lax.axis_index("s")

    @pl.when(sid == 0)
    def _():
        pltpu.sync_copy(zero_hbm, acc)

    def fill_zero(i, _):
        r = i // (DH // 16)
        j = i % (DH // 16)
        tr[r, pl.ds(DH + j * 16, 16)] = jnp.zeros((16,), jnp.float32)
        return 0

    lax.fori_loop(0, K * (DH // 16), fill_zero, 0)
    plsc.subcore_barrier()

    base = sid * EW
    gbase = (sid & 3) * G
    col0 = cid * DH

    def chunk(g, _):
        e0 = base + g * K
        pltpu.sync_copy(dst_hbm.at[pl.ds(e0, K)], dsti)
        pltpu.sync_copy(src_hbm.at[pl.ds(e0, K)], srci)
        pltpu.async_copy(a_hbm.at[dsti], ar, sem).wait()
        pltpu.async_copy(b_hbm.at[srci], br, sem).wait()
        pltpu.sync_copy(c_hbm.at[cid].at[pl.ds(e0, K)], cr)

        def comp(r, _):
            for j in range(DH // 16):
                sl = pl.ds(j * 16, 16)
                sf = pl.ds(col0 + j * 16, 16)
                tr[r, sl] = jnp.maximum(ar[r, sf] + br[r, sf] + cr[r, sl],
                                        0.0)
            return 0

        lax.fori_loop(0, K, comp, 0)
        pltpu.sync_copy(gidx_hbm.at[pl.ds(e0, K)], gidx)

        def gmap(m, _):
            sl = pl.ds(m * 16, 16)
            gidx[sl] = gidx[sl] + gbase
            return 0

        lax.fori_loop(0, K // 16, gmap, 0)
        pltpu.sync_copy(tr, acc.at[gidx], add=True)
        return 0

    lax.fori_loop(0, NCHUNK, chunk, 0)
    plsc.subcore_barrier()

    @pl.when(sid == 0)
    def _():
        pltpu.sync_copy(acc, out_hbm.at[cid])


def _t1_body(x_ref, wn1_ref, bn1_ref, wn2_ref, bn2_ref, we2_ref, be2_ref,
             wcs_ref, b1s_ref, wa_ref, wb_ref, batch_ref,
             h_ref, a_ref, b_ref, cs_ref, c0s_ref, starts_ref):
    t = jnp.maximum(x_ref[...] @ wn1_ref[...] + bn1_ref[...], 0.0)
    h = t @ wn2_ref[...] + bn2_ref[...]
    h_ref[...] = h
    oh = (lax.broadcasted_iota(jnp.int32, (G, N), 0) == batch_ref[...]).astype(
        jnp.float32)
    counts = oh @ jnp.ones((N, 1), jnp.float32)
    tri = (lax.broadcasted_iota(jnp.int32, (G, G), 0)
           > lax.broadcasted_iota(jnp.int32, (G, G), 1)).astype(jnp.float32)
    starts_ref[...] = (tri @ counts).astype(jnp.int32)
    cs_ref[...] = we2_ref[...] @ wcs_ref[...]
    c0s = be2_ref[...] @ wcs_ref[...] + b1s_ref[...]
    c0s_ref[...] = c0s
    a_ref[...] = h @ wa_ref[...] + c0s[:, 0:D]
    b_ref[...] = h @ wb_ref[...]


_t1_call = pl.pallas_call(
    _t1_body,
    out_shape=[
        jax.ShapeDtypeStruct((N, D), jnp.float32),
        jax.ShapeDtypeStruct((N, D), jnp.float32),
        jax.ShapeDtypeStruct((N, D), jnp.float32),
        jax.ShapeDtypeStruct((D, 3 * D), jnp.float32),
        jax.ShapeDtypeStruct((1, 3 * D), jnp.float32),
        jax.ShapeDtypeStruct((G, 1), jnp.int32),
    ],
)


EBLK = 4000


def _te_body(ea_ref, we1_ref, be1_ref, cs_ref, dst_ref, starts_ref,
             c01_ref, c2_ref, gidx_ref):
    t = jnp.maximum(ea_ref[...] @ we1_ref[...] + be1_ref[...], 0.0)
    cc = t @ cs_ref[...]
    for l in range(2):
        c01_ref[l] = cc[:, l * D:(l + 1) * D]
    c2_ref[0] = cc[:, 2 * D:2 * D + DH]
    c2_ref[1] = cc[:, 2 * D + DH:3 * D]
    ge = (dst_ref[0] >= starts_ref[...]).astype(jnp.int32)
    gidx_ref[0] = jnp.sum(ge, axis=0, keepdims=True) - 1


_te_call = pl.pallas_call(
    _te_body,
    grid=(E // EBLK,),
    in_specs=[
        pl.BlockSpec((EBLK, 16), lambda i: (i, 0)),
        pl.BlockSpec((16, D), lambda i: (0, 0)),
        pl.BlockSpec((1, D), lambda i: (0, 0)),
        pl.BlockSpec((D, 3 * D), lambda i: (0, 0)),
        pl.BlockSpec((1, 1, EBLK), lambda i: (i, 0, 0)),
        pl.BlockSpec((G, 1), lambda i: (0, 0)),
    ],
    out_specs=[
        pl.BlockSpec((2, EBLK, D), lambda i: (0, i, 0)),
        pl.BlockSpec((NC, EBLK, DH), lambda i: (0, i, 0)),
        pl.BlockSpec((1, 1, EBLK), lambda i: (i, 0, 0)),
    ],
    out_shape=[
        jax.ShapeDtypeStruct((2, E, D), jnp.float32),
        jax.ShapeDtypeStruct((NC, E, DH), jnp.float32),
        jax.ShapeDtypeStruct((E // EBLK, 1, EBLK), jnp.int32),
    ],
)


def _tpost_body(h_ref, sp_ref, sh_ref, deg_ref, w2_ref, b2_ref, wa_ref,
                c0_ref, wb_ref, hn_ref, a_ref, b_ref):
    s = jnp.concatenate(
        [sp_ref[0, 0:NR], sp_ref[1, 0:NR], sh_ref[0, 0:NR], sh_ref[1, 0:NR]],
        axis=0)
    aggr = s @ w2_ref[...] + deg_ref[...] * b2_ref[...]
    hn = jnp.maximum((2.0 + EPS) * h_ref[...] + aggr, 0.0)
    hn_ref[...] = hn
    a_ref[...] = hn @ wa_ref[...] + c0_ref[...]
    b_ref[...] = hn @ wb_ref[...]


_tpost_call = pl.pallas_call(
    _tpost_body,
    out_shape=[
        jax.ShapeDtypeStruct((N, D), jnp.float32),
        jax.ShapeDtypeStruct((N, D), jnp.float32),
        jax.ShapeDtypeStruct((N, D), jnp.float32),
    ],
)


def _tlast_body(h_ref, s_ref, deg_ref, w2_ref, b2_ref, batch_ref, out_ref):
    sp0 = jnp.sum(s_ref[0, :, 0:DH].reshape(4, G, DH), axis=0)
    sp1 = jnp.sum(s_ref[1, :, 0:DH].reshape(4, G, DH), axis=0)
    sp = jnp.concatenate([sp0, sp1], axis=-1)
    hf = (1.0 + EPS) * h_ref[...] + deg_ref[...] * b2_ref[...]
    oh = (lax.broadcasted_iota(jnp.int32, (G, N), 0) == batch_ref[...]).astype(
        jnp.float32)
    out_ref[...] = oh @ hf + sp @ w2_ref[...]


_tlast_call = pl.pallas_call(
    _tlast_body,
    out_shape=jax.ShapeDtypeStruct((G, D), jnp.float32),
)


def kernel(x, edge_attr, params, edge_index, batch):
    pn = params["init_node"]
    pe = params["init_edge"]
    convs = params["convs"]
    src = edge_index[0]
    dst = edge_index[1]

    wa = [p["W1"][0:D] for p in convs]
    wb = [p["W1"][D:2 * D] for p in convs]
    wcs = jnp.concatenate([p["W1"][2 * D:3 * D] for p in convs], axis=1)
    b1s = jnp.concatenate([p["b1"] for p in convs]).reshape(1, 3 * D)

    degs = _deg_kernel(dst)
    deg = (degs[0] + degs[1]).reshape(N, 1)

    h, a, b, cs, c0s, starts = _t1_call(
        x, pn["W1"], pn["b1"].reshape(1, D), pn["W2"], pn["b2"].reshape(1, D),
        pe["W2"], pe["b2"].reshape(1, D), wcs, b1s, wa[0], wb[0],
        batch.reshape(1, N))

    c01, c2, gidx = _te_call(edge_attr, pe["W1"], pe["b1"].reshape(1, D), cs,
                             dst.reshape(E // EBLK, 1, EBLK), starts)
    gidx = gidx.reshape(E)

    lsel_all = jnp.array(
        [[it >> 1, it & 1] + [0] * 14 for it in range(4)], jnp.int32)
    w2s = jnp.stack([convs[0]["W2"], convs[1]["W2"]])
    b2s = jnp.stack([convs[0]["b2"].reshape(1, D),
                     convs[1]["b2"].reshape(1, D)])
    was = jnp.stack([wa[1], wa[2]])
    wbs = jnp.stack([wb[1], wb[2]])
    c0n = jnp.stack([c0s[:, D:2 * D], c0s[:, 2 * D:3 * D]])
    zq = jnp.zeros((NACC, D), jnp.float32)

    nl = dst[0] * 0 + 4

    def cond(st):
        return st[0] < nl

    def body(st):
        it, hc, ac, bc, sprev = st
        l = it >> 1
        lsel = lax.dynamic_slice(lsel_all, (it, 0), (1, 16)).reshape(16)
        sh = _edge_kernel(ac, bc, c01, lsel, zq, dst, src)

        def do_post(op):
            hc_, ac_, bc_, sprev_, sh_ = op
            return _tpost_call(
                hc_, sprev_, sh_, deg,
                lax.dynamic_index_in_dim(w2s, l, keepdims=False),
                lax.dynamic_index_in_dim(b2s, l, keepdims=False),
                lax.dynamic_index_in_dim(was, l, keepdims=False),
                lax.dynamic_index_in_dim(c0n, l, keepdims=False),
                lax.dynamic_index_in_dim(wbs, l, keepdims=False))

        hn, an, bn = lax.cond(
            (it & 1) == 1, do_post, lambda op: (op[0], op[1], op[2]),
            (hc, ac, bc, sprev, sh))
        return (it + 1, hn, an, bn, sh)

    s0 = jnp.zeros((NC, NACC, D), jnp.float32)
    _, h, a, b, _ = lax.while_loop(
        cond, body, (jnp.int32(0), h, a, b, s0))

    zp = jnp.zeros((4 * G, D), jnp.float32)
    sp = _edge_pool_kernel(a, b, c2, dst, src, gidx, zp)
    out = _tlast_call(h, sp, deg, convs[2]["W2"],
                      convs[2]["b2"].reshape(1, D), batch.reshape(1, N))
    return out


# speedup vs baseline: 1.3935x; 1.3935x over previous
import functools

import jax
import jax.numpy as jnp
from jax import lax
from jax.experimental import pallas as pl
from jax.experimental.pallas import tpu as pltpu
from jax.experimental.pallas import tpu_sc as plsc

N = 10000
E = 320000
D = 128
G = 64
EPS = 0.1

NC = 2
NS = 16
DH = D // NC
EW = E // NS
K = 80
NCHUNK = EW // K
NR = N // 4
NACC = NR + NS

_mesh = plsc.VectorSubcoreMesh(core_axis_name="c", subcore_axis_name="s")


@functools.partial(
    pl.kernel,
    out_type=jax.ShapeDtypeStruct((NC, N), jnp.float32),
    mesh=_mesh,
    scratch_types=[
        pltpu.VMEM((K,), jnp.int32),
        pltpu.VMEM((K,), jnp.float32),
        pltpu.VMEM((N,), jnp.float32),
        pltpu.VMEM_SHARED((N,), jnp.float32),
    ],
)
def _deg_kernel(dst_hbm, out_hbm, idx_v, ones_v, stage_v, acc):
    cid = lax.axis_index("c")
    sid = lax.axis_index("s")

    def fill_ones(i, _):
        ones_v[pl.ds(i * 16, 16)] = jnp.full((16,), 1.0, jnp.float32)
        return 0

    lax.fori_loop(0, K // 16, fill_ones, 0)

    def fill_zero(i, _):
        stage_v[pl.ds(i * 16, 16)] = jnp.zeros((16,), jnp.float32)
        return 0

    lax.fori_loop(0, N // 16, fill_zero, 0)

    @pl.when(sid == 0)
    def _():
        pltpu.sync_copy(stage_v, acc)

    plsc.subcore_barrier()

    base = (cid * NS + sid) * (E // (NC * NS))

    def chunk(g, _):
        pltpu.sync_copy(dst_hbm.at[pl.ds(base + g * K, K)], idx_v)
        pltpu.sync_copy(ones_v, acc.at[idx_v], add=True)
        return 0

    lax.fori_loop(0, E // (NC * NS) // K, chunk, 0)
    plsc.subcore_barrier()

    @pl.when(sid == 0)
    def _():
        pltpu.sync_copy(acc, stage_v)
        pltpu.sync_copy(stage_v, out_hbm.at[cid])


@functools.partial(
    pl.kernel,
    out_type=jax.ShapeDtypeStruct((NC, NACC, D), jnp.float32),
    mesh=_mesh,
    scratch_types=[
        pltpu.VMEM((16,), jnp.int32),
        pltpu.VMEM((K,), jnp.int32),
        pltpu.VMEM((K,), jnp.int32),
        pltpu.VMEM((K,), jnp.int32),
        pltpu.VMEM((K, D), jnp.float32),
        pltpu.VMEM((K, D), jnp.float32),
        pltpu.VMEM((K, D), jnp.float32),
        pltpu.VMEM((K, D), jnp.float32),
        pltpu.VMEM_SHARED((NACC, D), jnp.float32),
        pltpu.SemaphoreType.DMA,
    ],
)
def _edge_kernel(a_hbm, b_hbm, c01_hbm, lsel_hbm, zero_hbm, dst_hbm,
                 src_hbm, out_hbm, lv, dsti, srci, dsts, ar, br, cr, tr, acc,
                 sem):
    cid = lax.axis_index("c")
    sid = lax.axis_index("s")

    pltpu.sync_copy(lsel_hbm, lv)
    lvv = lv[...]
    lsel = lvv[0]
    nhalf = lvv[1]

    @pl.when(sid == 0)
    def _():
        pltpu.sync_copy(zero_hbm, acc)

    plsc.subcore_barrier()

    base = sid * EW
    node0 = nhalf * (2 * NR) + cid * NR
    trash = NR + sid

    def chunk(g, _):
        e0 = base + g * K
        cpd = pltpu.async_copy(dst_hbm.at[pl.ds(e0, K)], dsti, sem)
        cps = pltpu.async_copy(src_hbm.at[pl.ds(e0, K)], srci, sem)
        cpd.wait()
        cps.wait()
        ga = pltpu.async_copy(a_hbm.at[dsti], ar, sem)
        gb = pltpu.async_copy(b_hbm.at[srci], br, sem)
        cpc = pltpu.async_copy(c01_hbm.at[lsel].at[pl.ds(e0, K)], cr, sem)

        def remap(m, _):
            sl = pl.ds(m * 16, 16)
            k = dsti[sl] - node0
            ok = (k >= 0) & (k < NR)
            dsts[sl] = jnp.where(ok, k, trash)
            return 0

        lax.fori_loop(0, K // 16, remap, 0)
        ga.wait()
        gb.wait()
        cpc.wait()

        def comp(r, _):
            for j in range(D // 16):
                sl = pl.ds(j * 16, 16)
                tr[r, sl] = jnp.maximum(ar[r, sl] + br[r, sl] + cr[r, sl],
                                        0.0)
            return 0

        lax.fori_loop(0, K, comp, 0)
        pltpu.sync_copy(tr, acc.at[dsts], add=True)
        return 0

    lax.fori_loop(0, NCHUNK, chunk, 0)
    plsc.subcore_barrier()

    @pl.when(sid == 0)
    def _():
        pltpu.sync_copy(acc, out_hbm.at[cid])


@functools.partial(
    pl.kernel,
    out_type=jax.ShapeDtypeStruct((NC, 4 * G, D), jnp.float32),
    mesh=_mesh,
    scratch_types=[
        pltpu.VMEM((K,), jnp.int32),
        pltpu.VMEM((K,), jnp.int32),
        pltpu.VMEM((K,), jnp.int32),
        pltpu.VMEM((K, D), jnp.float32),
        pltpu.VMEM((K, D), jnp.float32),
        pltpu.VMEM((K, DH), jnp.float32),
        pltpu.VMEM((K, D), jnp.float32),
        pltpu.VMEM_SHARED((4 * G, D), jnp.float32),
        pltpu.SemaphoreType.DMA,
    ],
)
def _edge_pool_kernel(a_hbm, b_hbm, c_hbm, dst_hbm, src_hbm, gidx_hbm,
                      zero_hbm, out_hbm, dsti, srci, gidx, ar, br, cr, tr,
                      acc, sem):
    cid = lax.axis_index("c")
    sid = lax.axis_index("s")

    @pl.when(sid == 0)
    def _():
        pltpu.sync_copy(zero_hbm, acc)

    def fill_zero(i, _):
        r = i // (DH // 16)
        j = i % (DH // 16)
        tr[r, pl.ds(DH + j * 16, 16)] = jnp.zeros((16,), jnp.float32)
        return 0

    lax.fori_loop(0, K * (DH // 16), fill_zero, 0)
    plsc.subcore_barrier()

    base = sid * EW
    gbase = (sid & 3) * G
    col0 = cid * DH

    def chunk(g, _):
        e0 = base + g * K
        cpd = pltpu.async_copy(dst_hbm.at[pl.ds(e0, K)], dsti, sem)
        cps = pltpu.async_copy(src_hbm.at[pl.ds(e0, K)], srci, sem)
        cpd.wait()
        cps.wait()
        ga = pltpu.async_copy(a_hbm.at[dsti], ar, sem)
        gb = pltpu.async_copy(b_hbm.at[srci], br, sem)
        cpc = pltpu.async_copy(c_hbm.at[cid].at[pl.ds(e0, K)], cr, sem)
        cpg = pltpu.async_copy(gidx_hbm.at[pl.ds(e0, K)], gidx, sem)
        ga.wait()
        gb.wait()
        cpc.wait()
        cpg.wait()

        def comp(r, _):
            for j in range(DH // 16):
                sl = pl.ds(j * 16, 16)
                sf = pl.ds(col0 + j * 16, 16)
                tr[r, sl] = jnp.maximum(ar[r, sf] + br[r, sf] + cr[r, sl],
                                        0.0)
            return 0

        lax.fori_loop(0, K, comp, 0)

        def gmap(m, _):
            sl = pl.ds(m * 16, 16)
            gidx[sl] = gidx[sl] + gbase
            return 0

        lax.fori_loop(0, K // 16, gmap, 0)
        pltpu.sync_copy(tr, acc.at[gidx], add=True)
        return 0

    lax.fori_loop(0, NCHUNK, chunk, 0)
    plsc.subcore_barrier()

    @pl.when(sid == 0)
    def _():
        pltpu.sync_copy(acc, out_hbm.at[cid])


def _t1_body(x_ref, wn1_ref, bn1_ref, wn2_ref, bn2_ref, we2_ref, be2_ref,
             wcs_ref, b1s_ref, wa_ref, wb_ref, batch_ref,
             h_ref, a_ref, b_ref, cs_ref, c0s_ref, starts_ref):
    t = jnp.maximum(x_ref[...] @ wn1_ref[...] + bn1_ref[...], 0.0)
    h = t @ wn2_ref[...] + bn2_ref[...]
    h_ref[...] = h
    oh = (lax.broadcasted_iota(jnp.int32, (G, N), 0) == batch_ref[...]).astype(
        jnp.float32)
    counts = oh @ jnp.ones((N, 1), jnp.float32)
    tri = (lax.broadcasted_iota(jnp.int32, (G, G), 0)
           > lax.broadcasted_iota(jnp.int32, (G, G), 1)).astype(jnp.float32)
    starts_ref[...] = (tri @ counts).astype(jnp.int32)
    cs_ref[...] = we2_ref[...] @ wcs_ref[...]
    c0s = be2_ref[...] @ wcs_ref[...] + b1s_ref[...]
    c0s_ref[...] = c0s
    a_ref[...] = h @ wa_ref[...] + c0s[:, 0:D]
    b_ref[...] = h @ wb_ref[...]


_t1_call = pl.pallas_call(
    _t1_body,
    out_shape=[
        jax.ShapeDtypeStruct((N, D), jnp.float32),
        jax.ShapeDtypeStruct((N, D), jnp.float32),
        jax.ShapeDtypeStruct((N, D), jnp.float32),
        jax.ShapeDtypeStruct((D, 3 * D), jnp.float32),
        jax.ShapeDtypeStruct((1, 3 * D), jnp.float32),
        jax.ShapeDtypeStruct((G, 1), jnp.int32),
    ],
)


EBLK = 4000


def _te_body(ea_ref, we1_ref, be1_ref, cs_ref, dst_ref, starts_ref,
             c01_ref, c2_ref, gidx_ref):
    t = jnp.maximum(ea_ref[...] @ we1_ref[...] + be1_ref[...], 0.0)
    cc = t @ cs_ref[...]
    for l in range(2):
        c01_ref[l] = cc[:, l * D:(l + 1) * D]
    c2_ref[0] = cc[:, 2 * D:2 * D + DH]
    c2_ref[1] = cc[:, 2 * D + DH:3 * D]
    ge = (dst_ref[0] >= starts_ref[...]).astype(jnp.int32)
    gidx_ref[0] = jnp.sum(ge, axis=0, keepdims=True) - 1


_te_call = pl.pallas_call(
    _te_body,
    grid=(E // EBLK,),
    in_specs=[
        pl.BlockSpec((EBLK, 16), lambda i: (i, 0)),
        pl.BlockSpec((16, D), lambda i: (0, 0)),
        pl.BlockSpec((1, D), lambda i: (0, 0)),
        pl.BlockSpec((D, 3 * D), lambda i: (0, 0)),
        pl.BlockSpec((1, 1, EBLK), lambda i: (i, 0, 0)),
        pl.BlockSpec((G, 1), lambda i: (0, 0)),
    ],
    out_specs=[
        pl.BlockSpec((2, EBLK, D), lambda i: (0, i, 0)),
        pl.BlockSpec((NC, EBLK, DH), lambda i: (0, i, 0)),
        pl.BlockSpec((1, 1, EBLK), lambda i: (i, 0, 0)),
    ],
    out_shape=[
        jax.ShapeDtypeStruct((2, E, D), jnp.float32),
        jax.ShapeDtypeStruct((NC, E, DH), jnp.float32),
        jax.ShapeDtypeStruct((E // EBLK, 1, EBLK), jnp.int32),
    ],
)


def _tpost_body(h_ref, sp_ref, sh_ref, deg_ref, w2_ref, b2_ref, wa_ref,
                c0_ref, wb_ref, hn_ref, a_ref, b_ref):
    s = jnp.concatenate(
        [sp_ref[0, 0:NR], sp_ref[1, 0:NR], sh_ref[0, 0:NR], sh_ref[1, 0:NR]],
        axis=0)
    aggr = s @ w2_ref[...] + deg_ref[...] * b2_ref[...]
    hn = jnp.maximum((2.0 + EPS) * h_ref[...] + aggr, 0.0)
    hn_ref[...] = hn
    a_ref[...] = hn @ wa_ref[...] + c0_ref[...]
    b_ref[...] = hn @ wb_ref[...]


_tpost_call = pl.pallas_call(
    _tpost_body,
    out_shape=[
        jax.ShapeDtypeStruct((N, D), jnp.float32),
        jax.ShapeDtypeStruct((N, D), jnp.float32),
        jax.ShapeDtypeStruct((N, D), jnp.float32),
    ],
)


def _tlast_body(h_ref, s_ref, deg_ref, w2_ref, b2_ref, batch_ref, out_ref):
    sp0 = jnp.sum(s_ref[0, :, 0:DH].reshape(4, G, DH), axis=0)
    sp1 = jnp.sum(s_ref[1, :, 0:DH].reshape(4, G, DH), axis=0)
    sp = jnp.concatenate([sp0, sp1], axis=-1)
    hf = (1.0 + EPS) * h_ref[...] + deg_ref[...] * b2_ref[...]
    oh = (lax.broadcasted_iota(jnp.int32, (G, N), 0) == batch_ref[...]).astype(
        jnp.float32)
    out_ref[...] = oh @ hf + sp @ w2_ref[...]


_tlast_call = pl.pallas_call(
    _tlast_body,
    out_shape=jax.ShapeDtypeStruct((G, D), jnp.float32),
)


def kernel(x, edge_attr, params, edge_index, batch):
    pn = params["init_node"]
    pe = params["init_edge"]
    convs = params["convs"]
    src = edge_index[0]
    dst = edge_index[1]

    wa = [p["W1"][0:D] for p in convs]
    wb = [p["W1"][D:2 * D] for p in convs]
    wcs = jnp.concatenate([p["W1"][2 * D:3 * D] for p in convs], axis=1)
    b1s = jnp.concatenate([p["b1"] for p in convs]).reshape(1, 3 * D)

    degs = _deg_kernel(dst)
    deg = (degs[0] + degs[1]).reshape(N, 1)

    h, a, b, cs, c0s, starts = _t1_call(
        x, pn["W1"], pn["b1"].reshape(1, D), pn["W2"], pn["b2"].reshape(1, D),
        pe["W2"], pe["b2"].reshape(1, D), wcs, b1s, wa[0], wb[0],
        batch.reshape(1, N))

    c01, c2, gidx = _te_call(edge_attr, pe["W1"], pe["b1"].reshape(1, D), cs,
                             dst.reshape(E // EBLK, 1, EBLK), starts)
    gidx = gidx.reshape(E)

    lsel_all = jnp.array(
        [[it >> 1, it & 1] + [0] * 14 for it in range(4)], jnp.int32)
    w2s = jnp.stack([convs[0]["W2"], convs[1]["W2"]])
    b2s = jnp.stack([convs[0]["b2"].reshape(1, D),
                     convs[1]["b2"].reshape(1, D)])
    was = jnp.stack([wa[1], wa[2]])
    wbs = jnp.stack([wb[1], wb[2]])
    c0n = jnp.stack([c0s[:, D:2 * D], c0s[:, 2 * D:3 * D]])
    zq = jnp.zeros((NACC, D), jnp.float32)

    nl = dst[0] * 0 + 4

    def cond(st):
        return st[0] < nl

    def body(st):
        it, hc, ac, bc, sprev = st
        l = it >> 1
        lsel = lax.dynamic_slice(lsel_all, (it, 0), (1, 16)).reshape(16)
        sh = _edge_kernel(ac, bc, c01, lsel, zq, dst, src)

        def do_post(op):
            hc_, ac_, bc_, sprev_, sh_ = op
            return _tpost_call(
                hc_, sprev_, sh_, deg,
                lax.dynamic_index_in_dim(w2s, l, keepdims=False),
                lax.dynamic_index_in_dim(b2s, l, keepdims=False),
                lax.dynamic_index_in_dim(was, l, keepdims=False),
                lax.dynamic_index_in_dim(c0n, l, keepdims=False),
                lax.dynamic_index_in_dim(wbs, l, keepdims=False))

        hn, an, bn = lax.cond(
            (it & 1) == 1, do_post, lambda op: (op[0], op[1], op[2]),
            (hc, ac, bc, sprev, sh))
        return (it + 1, hn, an, bn, sh)

    s0 = jnp.zeros((NC, NACC, D), jnp.float32)
    _, h, a, b, _ = lax.while_loop(
        cond, body, (jnp.int32(0), h, a, b, s0))

    zp = jnp.zeros((4 * G, D), jnp.float32)
    sp = _edge_pool_kernel(a, b, c2, dst, src, gidx, zp)
    out = _tlast_call(h, sp, deg, convs[2]["W2"],
                      convs[2]["b2"].reshape(1, D), batch.reshape(1, N))
    return out
